# baseline (speedup 1.0000x reference)
import functools

import jax
import jax.numpy as jnp
from jax import lax
from jax.experimental import pallas as pl
from jax.experimental.pallas import tpu as pltpu
from jax.experimental.pallas import tpu_sc as plsc

_B = 1024
_F = 4 * 64 * 64
_TPAD = 1024
_FB = 2048

_NW = 32
_BPW = _B // _NW


def _sc_gather(t_hbm, a_hbm, b_hbm, av_hbm, bv_hbm,
               idx_v, avo_v, bvo_v, sem):
    wid = lax.axis_index("s") * 2 + lax.axis_index("c")
    base = wid * _BPW
    pltpu.sync_copy(t_hbm.at[pl.ds(base, _BPW)], idx_v)
    pltpu.async_copy(a_hbm.at[idx_v], avo_v, sem).wait()
    pltpu.async_copy(b_hbm.at[idx_v], bvo_v, sem).wait()
    pltpu.sync_copy(avo_v, av_hbm.at[pl.ds(base, _BPW)])
    pltpu.sync_copy(bvo_v, bv_hbm.at[pl.ds(base, _BPW)])


def _gather_coeffs(timestep, ap, bp):
    mesh = plsc.VectorSubcoreMesh(core_axis_name="c", subcore_axis_name="s")
    run = functools.partial(
        pl.kernel,
        mesh=mesh,
        out_type=[
            jax.ShapeDtypeStruct((_B,), jnp.float32),
            jax.ShapeDtypeStruct((_B,), jnp.float32),
        ],
        scratch_types=[
            pltpu.VMEM((_BPW,), jnp.int32),
            pltpu.VMEM((_BPW,), jnp.float32),
            pltpu.VMEM((_BPW,), jnp.float32),
            pltpu.SemaphoreType.DMA,
        ],
    )(_sc_gather)
    return run(timestep, ap, bp)


def _tc_body(av_ref, bv_ref, x_ref, n_ref, o_ref):
    o_ref[...] = av_ref[...] * x_ref[...] + bv_ref[...] * n_ref[...]


def kernel(x_0, timestep, noise, a, b):
    x2 = x_0.transpose(1, 2, 3, 0).reshape(_F, _B)
    n2 = noise.transpose(1, 2, 3, 0).reshape(_F, _B)
    t1 = timestep.astype(jnp.int32)
    ap = jnp.pad(a, (0, _TPAD - a.shape[0]))
    bp = jnp.pad(b, (0, _TPAD - b.shape[0]))

    av, bv = _gather_coeffs(t1, ap, bp)
    av2 = av.reshape(1, _B)
    bv2 = bv.reshape(1, _B)

    grid = (_F // _FB,)
    out = pl.pallas_call(
        _tc_body,
        grid=grid,
        in_specs=[
            pl.BlockSpec((1, _B), lambda i: (0, 0)),
            pl.BlockSpec((1, _B), lambda i: (0, 0)),
            pl.BlockSpec((_FB, _B), lambda i: (i, 0)),
            pl.BlockSpec((_FB, _B), lambda i: (i, 0)),
        ],
        out_specs=pl.BlockSpec((_FB, _B), lambda i: (i, 0)),
        out_shape=jax.ShapeDtypeStruct((_F, _B), jnp.float32),
        compiler_params=pltpu.CompilerParams(
            dimension_semantics=("arbitrary",),
        ),
    )(av2, bv2, x2, n2)
    return out.reshape(4, 64, 64, _B).transpose(3, 0, 1, 2)


# speedup vs baseline: 1.2461x; 1.2461x over previous
import jax
import jax.numpy as jnp
from jax.experimental import pallas as pl
from jax.experimental.pallas import tpu as pltpu

_B = 1024
_F = 4 * 64 * 64
_TPAD = 1024
_HB = 1024


def _body(t_ref, a_ref, b_ref, xa_ref, xb_ref, na_ref, nb_ref, o_ref,
          av_ref, bv_ref):
    @pl.when(pl.program_id(0) == 0)
    def _gather():
        iota = jax.lax.broadcasted_iota(jnp.int32, (_TPAD, _B), 0)
        oh = iota == t_ref[...]
        av_ref[...] = jnp.sum(jnp.where(oh, a_ref[...], 0.0), axis=0,
                              keepdims=True)
        bv_ref[...] = jnp.sum(jnp.where(oh, b_ref[...], 0.0), axis=0,
                              keepdims=True)

    av = av_ref[...]
    bv = bv_ref[...]
    o_ref[0:_HB, :] = av * xa_ref[...] + bv * na_ref[...]
    o_ref[_HB:2 * _HB, :] = av * xb_ref[...] + bv * nb_ref[...]


def kernel(x_0, timestep, noise, a, b):
    x2 = x_0.transpose(1, 2, 3, 0).reshape(_F, _B)
    n2 = noise.transpose(1, 2, 3, 0).reshape(_F, _B)
    t2 = timestep.reshape(1, _B).astype(jnp.int32)
    ap = jnp.pad(a, (0, _TPAD - a.shape[0])).reshape(_TPAD, 1)
    bp = jnp.pad(b, (0, _TPAD - b.shape[0])).reshape(_TPAD, 1)

    grid = (_F // (2 * _HB),)
    out = pl.pallas_call(
        _body,
        grid=grid,
        in_specs=[
            pl.BlockSpec((1, _B), lambda i: (0, 0)),
            pl.BlockSpec((_TPAD, 1), lambda i: (0, 0)),
            pl.BlockSpec((_TPAD, 1), lambda i: (0, 0)),
            pl.BlockSpec((_HB, _B), lambda i: (2 * i, 0)),
            pl.BlockSpec((_HB, _B), lambda i: (2 * i + 1, 0)),
            pl.BlockSpec((_HB, _B), lambda i: (2 * i, 0)),
            pl.BlockSpec((_HB, _B), lambda i: (2 * i + 1, 0)),
        ],
        out_specs=pl.BlockSpec((2 * _HB, _B), lambda i: (i, 0)),
        out_shape=jax.ShapeDtypeStruct((_F, _B), jnp.float32),
        scratch_shapes=[
            pltpu.VMEM((1, _B), jnp.float32),
            pltpu.VMEM((1, _B), jnp.float32),
        ],
        compiler_params=pltpu.CompilerParams(
            dimension_semantics=("arbitrary",),
        ),
    )(t2, ap, bp, x2, x2, n2, n2)
    return out.reshape(4, 64, 64, _B).transpose(3, 0, 1, 2)


# speedup vs baseline: 1.3320x; 1.0689x over previous
import jax
import jax.numpy as jnp
from jax.experimental import pallas as pl
from jax.experimental.pallas import tpu as pltpu

_B = 1024
_F = 4 * 64 * 64
_NT = 1000
_HB = 1024


def _body(t_ref, a_ref, b_ref, xa_ref, xb_ref, na_ref, nb_ref, o_ref,
          av_ref, bv_ref):
    @pl.when(pl.program_id(0) == 0)
    def _gather():
        t_row = t_ref[...].reshape(1, _B)
        iota = jax.lax.broadcasted_iota(jnp.int32, (_NT, _B), 0)
        oh = (iota == t_row).astype(jnp.float32)
        a_row = a_ref[...].reshape(1, _NT)
        b_row = b_ref[...].reshape(1, _NT)
        av_ref[...] = jax.lax.dot(a_row, oh,
                                  preferred_element_type=jnp.float32)
        bv_ref[...] = jax.lax.dot(b_row, oh,
                                  preferred_element_type=jnp.float32)

    av = av_ref[...]
    bv = bv_ref[...]
    o_ref[0:_HB, :] = av * xa_ref[...] + bv * na_ref[...]
    o_ref[_HB:2 * _HB, :] = av * xb_ref[...] + bv * nb_ref[...]


def kernel(x_0, timestep, noise, a, b):
    x2 = x_0.transpose(1, 2, 3, 0).reshape(_F, _B)
    n2 = noise.transpose(1, 2, 3, 0).reshape(_F, _B)

    grid = (_F // (2 * _HB),)
    out = pl.pallas_call(
        _body,
        grid=grid,
        in_specs=[
            pl.BlockSpec((_B,), lambda i: (0,)),
            pl.BlockSpec((_NT,), lambda i: (0,)),
            pl.BlockSpec((_NT,), lambda i: (0,)),
            pl.BlockSpec((_HB, _B), lambda i: (2 * i, 0)),
            pl.BlockSpec((_HB, _B), lambda i: (2 * i + 1, 0)),
            pl.BlockSpec((_HB, _B), lambda i: (2 * i, 0)),
            pl.BlockSpec((_HB, _B), lambda i: (2 * i + 1, 0)),
        ],
        out_specs=pl.BlockSpec((2 * _HB, _B), lambda i: (i, 0)),
        out_shape=jax.ShapeDtypeStruct((_F, _B), jnp.float32),
        scratch_shapes=[
            pltpu.VMEM((1, _B), jnp.float32),
            pltpu.VMEM((1, _B), jnp.float32),
        ],
        compiler_params=pltpu.CompilerParams(
            dimension_semantics=("arbitrary",),
        ),
    )(timestep, a, b, x2, x2, n2, n2)
    return out.reshape(4, 64, 64, _B).transpose(3, 0, 1, 2)
